# baseline (speedup 1.0000x reference)
import functools

import jax
import jax.numpy as jnp
from jax import lax
from jax.experimental import pallas as pl
from jax.experimental.pallas import tpu as pltpu
from jax.experimental.pallas import tpu_sc as plsc

_N = 10000
_E = 320000
_D = 128
_NC = 2
_NS = 16
_NW = _NC * _NS
_EPW = _E // _NW
_NH = _N // _NC
_CAP = 5504
_CAPV = _CAP + 16
_NP = 5248
_RPT = _NP // _NS
_CH = 128
_NCHUNK = _NW * _CAP // _NS // _CH

_mesh = plsc.VectorSubcoreMesh(core_axis_name="c", subcore_axis_name="s")


@functools.partial(
    pl.kernel,
    out_type=(jax.ShapeDtypeStruct((_NC, _NW, _CAP), jnp.int32),
              jax.ShapeDtypeStruct((_NC, _NW, _CAP), jnp.int32)),
    mesh=_mesh,
    compiler_params=pltpu.CompilerParams(needs_layout_passes=False),
    scratch_types=[
        pltpu.VMEM((_EPW,), jnp.int32),
        pltpu.VMEM((_EPW,), jnp.int32),
        pltpu.VMEM((_CAPV,), jnp.int32),
        pltpu.VMEM((_CAPV,), jnp.int32),
        pltpu.VMEM((_CAPV,), jnp.int32),
        pltpu.VMEM((_CAPV,), jnp.int32),
    ],
)
def _sc_route(src_hbm, dst_hbm, bsrc_hbm, bdst_hbm,
              src_v, dst_v, b0s, b0d, b1s, b1d):
    c = lax.axis_index("c")
    s = lax.axis_index("s")
    w = c * _NS + s

    pltpu.sync_copy(src_hbm.at[pl.ds(w * _EPW, _EPW)], src_v)
    pltpu.sync_copy(dst_hbm.at[pl.ds(w * _EPW, _EPW)], dst_v)

    pad_s = jnp.zeros((16,), jnp.int32)
    pad_d = jnp.full((16,), _NH, jnp.int32)

    def fill(i, _):
        b0s[pl.ds(16 * i, 16)] = pad_s
        b0d[pl.ds(16 * i, 16)] = pad_d
        b1s[pl.ds(16 * i, 16)] = pad_s
        b1d[pl.ds(16 * i, 16)] = pad_d
        return 0

    lax.fori_loop(0, _CAPV // 16, fill, 0)

    def route(i, offs):
        o0, o1 = offs
        sv = src_v[pl.ds(16 * i, 16)]
        dv = dst_v[pl.ds(16 * i, 16)]
        dsort, ssort = plsc.sort_key_val(dv, sv)
        m0 = dsort < _NH
        n0 = plsc.all_reduce_population_count(m0)[0]
        b0s[pl.ds(o0, 16)] = ssort
        b0d[pl.ds(o0, 16)] = dsort
        srev = lax.rev(ssort, (0,))
        drev = lax.rev(dsort, (0,))
        b1s[pl.ds(o1, 16)] = srev
        b1d[pl.ds(o1, 16)] = drev - _NH
        o0 = jnp.minimum(o0 + n0, _CAP)
        o1 = jnp.minimum(o1 + (16 - n0), _CAP)
        return (o0, o1)

    o0, o1 = lax.fori_loop(0, _EPW // 16, route,
                           (jnp.int32(0), jnp.int32(0)))
    b0s[pl.ds(o0, 16)] = pad_s
    b0d[pl.ds(o0, 16)] = pad_d
    b1s[pl.ds(o1, 16)] = pad_s
    b1d[pl.ds(o1, 16)] = pad_d

    pltpu.sync_copy(b0s.at[pl.ds(0, _CAP)], bsrc_hbm.at[0, w])
    pltpu.sync_copy(b0d.at[pl.ds(0, _CAP)], bdst_hbm.at[0, w])
    pltpu.sync_copy(b1s.at[pl.ds(0, _CAP)], bsrc_hbm.at[1, w])
    pltpu.sync_copy(b1d.at[pl.ds(0, _CAP)], bdst_hbm.at[1, w])


@functools.partial(
    pl.kernel,
    out_type=jax.ShapeDtypeStruct((_NC, _NP, _D), jnp.float32),
    mesh=_mesh,
    scratch_types=[
        pltpu.VMEM((_NCHUNK, _CH), jnp.int32),
        pltpu.VMEM((_NCHUNK, _CH), jnp.int32),
        pltpu.VMEM((_CH, _D), jnp.float32),
        pltpu.VMEM((_CH, _D), jnp.float32),
        pltpu.VMEM_SHARED((_NP, _D), jnp.float32),
        pltpu.SemaphoreType.DMA,
        pltpu.SemaphoreType.DMA,
    ],
)
def _sc_gather_scatter(feat_hbm, src_hbm, dst_hbm, zeros_hbm, out_hbm,
                       src_v, dst_v, buf0, buf1, h_sh, sem0, sem1):
    c = lax.axis_index("c")
    s = lax.axis_index("s")
    r0 = s * _RPT

    pltpu.sync_copy(zeros_hbm.at[pl.ds(r0, _RPT)],
                    h_sh.at[pl.ds(r0, _RPT)])
    pltpu.sync_copy(src_hbm.at[c, s], src_v)
    pltpu.sync_copy(dst_hbm.at[c, s], dst_v)
    plsc.subcore_barrier()

    def wait_gather(buf, sem):
        pltpu.make_async_copy(feat_hbm.at[pl.ds(0, _CH)], buf, sem).wait()

    pltpu.async_copy(feat_hbm.at[src_v.at[0]], buf0, sem0)
    pltpu.async_copy(feat_hbm.at[src_v.at[1]], buf1, sem1)

    def step(k, _):
        j = 2 * k
        wait_gather(buf0, sem0)
        pltpu.sync_copy(buf0, h_sh.at[dst_v.at[j]], add=True)
        pltpu.async_copy(feat_hbm.at[src_v.at[j + 2]], buf0, sem0)
        wait_gather(buf1, sem1)
        pltpu.sync_copy(buf1, h_sh.at[dst_v.at[j + 1]], add=True)
        pltpu.async_copy(feat_hbm.at[src_v.at[j + 3]], buf1, sem1)
        return 0

    lax.fori_loop(0, _NCHUNK // 2 - 1, step, 0)
    wait_gather(buf0, sem0)
    pltpu.sync_copy(buf0, h_sh.at[dst_v.at[_NCHUNK - 2]], add=True)
    wait_gather(buf1, sem1)
    pltpu.sync_copy(buf1, h_sh.at[dst_v.at[_NCHUNK - 1]], add=True)

    plsc.subcore_barrier()
    pltpu.sync_copy(h_sh.at[pl.ds(r0, _RPT)],
                    out_hbm.at[c, pl.ds(r0, _RPT)])


def _lin_body(h_ref, w_ref, b_ref, o_ref):
    o_ref[...] = lax.dot_general(
        h_ref[0], w_ref[...], (((1,), (1,)), ((), ())),
        preferred_element_type=jnp.float32) + b_ref[...]


_linear = pl.pallas_call(
    _lin_body,
    grid=(10,),
    in_specs=[
        pl.BlockSpec((1, 1000, _D), lambda i: (i // 5, i % 5, 0)),
        pl.BlockSpec((_D, _D), lambda i: (0, 0)),
        pl.BlockSpec((1, _D), lambda i: (0, 0)),
    ],
    out_specs=pl.BlockSpec((1000, _D), lambda i: (i, 0)),
    out_shape=jax.ShapeDtypeStruct((_N, _D), jnp.float32),
)


@jax.jit
def kernel(features, edge_index, W, b):
    bsrc, bdst = _sc_route(edge_index[0], edge_index[1])
    bsrc = bsrc.reshape(_NC, _NS, _NCHUNK, _CH)
    bdst = bdst.reshape(_NC, _NS, _NCHUNK, _CH)
    zeros = jnp.zeros((_NP, _D), jnp.float32)
    hpart = _sc_gather_scatter(features, bsrc, bdst, zeros)
    return _linear(hpart, W, b.reshape(1, _D))
